# Initial kernel scaffold; baseline (speedup 1.0000x reference)
import jax
import jax.numpy as jnp
from jax.experimental import pallas as pl


def kernel(x, edge_index, W1, b1, g1, bt1, a1, W2, b2, g2, bt2, a2):
    raise NotImplementedError("write your pallas kernel here")


# speedup vs baseline: 8.1709x; 8.1709x over previous
import functools

import jax
import jax.numpy as jnp
from jax import lax
from jax.experimental import pallas as pl
from jax.experimental.pallas import tpu as pltpu
from jax.experimental.pallas import tpu_sc as plsc

_N = 10000
_D = 128
_E = 320000
_NPAD = 10240
_EPAD = 327680
_NW = 32
_CH = 80
_C = 128
_RT = _NPAD // 16
_EPS = 1e-5
_BR = 512

_mesh = plsc.VectorSubcoreMesh(core_axis_name="c", subcore_axis_name="s")


@functools.partial(
    pl.kernel,
    out_type=jax.ShapeDtypeStruct((2, _NPAD, _D), jnp.float32),
    mesh=_mesh,
    scratch_types=[
        pltpu.VMEM((_CH, _C), jnp.int32),
        pltpu.VMEM((_C, _D), jnp.float32),
        pltpu.VMEM_SHARED((_NPAD, _D), jnp.float32),
    ],
)
def _deg_kernel(dst_hbm, ones_hbm, zeros_hbm, out_hbm, didx, ones_v, deg_sh):
    c = lax.axis_index("c")
    s = lax.axis_index("s")
    wid = s * 2 + c
    pltpu.sync_copy(dst_hbm.at[wid], didx)
    pltpu.sync_copy(ones_hbm, ones_v)
    pltpu.sync_copy(zeros_hbm, deg_sh.at[pl.ds(s * _RT, _RT)])
    plsc.subcore_barrier()

    def body(j, carry):
        pltpu.sync_copy(ones_v, deg_sh.at[didx.at[j]], add=True)
        return carry

    lax.fori_loop(0, _CH, body, 0)
    plsc.subcore_barrier()
    pltpu.sync_copy(deg_sh.at[pl.ds(s * _RT, _RT)],
                    out_hbm.at[c, pl.ds(s * _RT, _RT)])


@functools.partial(
    pl.kernel,
    out_type=jax.ShapeDtypeStruct((2, _NPAD, _D), jnp.float32),
    mesh=_mesh,
    scratch_types=[
        pltpu.VMEM((_CH, _C), jnp.int32),
        pltpu.VMEM((_CH, _C), jnp.int32),
        pltpu.VMEM((_C, _D), jnp.float32),
        pltpu.VMEM_SHARED((_NPAD, _D), jnp.float32),
        pltpu.SemaphoreType.DMA,
    ],
)
def _agg_kernel(t_hbm, src_hbm, dst_hbm, zeros_hbm, out_hbm, sidx, didx,
                rows_v, agg_sh, sem):
    c = lax.axis_index("c")
    s = lax.axis_index("s")
    wid = s * 2 + c
    pltpu.sync_copy(src_hbm.at[wid], sidx)
    pltpu.sync_copy(dst_hbm.at[wid], didx)
    pltpu.sync_copy(zeros_hbm, agg_sh.at[pl.ds(s * _RT, _RT)])
    plsc.subcore_barrier()

    def body(j, carry):
        pltpu.async_copy(t_hbm.at[sidx.at[j]], rows_v, sem).wait()
        pltpu.sync_copy(rows_v, agg_sh.at[didx.at[j]], add=True)
        return carry

    lax.fori_loop(0, _CH, body, 0)
    plsc.subcore_barrier()
    pltpu.sync_copy(agg_sh.at[pl.ds(s * _RT, _RT)],
                    out_hbm.at[c, pl.ds(s * _RT, _RT)])


def _dinv_kernel(degp):
    def body(d_ref, o_ref):
        d = d_ref[...]
        deg = d[0, :, 0:1] + d[1, :, 0:1] + 1.0
        o_ref[...] = 1.0 / jnp.sqrt(deg)

    return pl.pallas_call(
        body,
        out_shape=jax.ShapeDtypeStruct((_NPAD, 1), jnp.float32),
    )(degp)


def _mm_scale(x, W, dinv):
    def body(x_ref, w_ref, dv_ref, o_ref):
        o_ref[...] = jnp.dot(x_ref[...], w_ref[...],
                             preferred_element_type=jnp.float32) * dv_ref[...]

    return pl.pallas_call(
        body,
        grid=(_NPAD // _BR,),
        in_specs=[
            pl.BlockSpec((_BR, _D), lambda i: (i, 0)),
            pl.BlockSpec((_D, _D), lambda i: (0, 0)),
            pl.BlockSpec((_BR, 1), lambda i: (i, 0)),
        ],
        out_specs=pl.BlockSpec((_BR, _D), lambda i: (i, 0)),
        out_shape=jax.ShapeDtypeStruct((_NPAD, _D), jnp.float32),
    )(x, W, dinv)


def _combine_stats(agg, t, dinv, b):
    def body(a0_ref, a1_ref, t_ref, dv_ref, b_ref, y_ref, st_ref):
        i = pl.program_id(0)
        y = dv_ref[...] * (a0_ref[...][0] + a1_ref[...][0] + t_ref[...]) \
            + b_ref[...]
        y_ref[...] = y
        rid = lax.broadcasted_iota(jnp.int32, (_BR, 1), 0) + i * _BR
        m = (rid < _N).astype(jnp.float32)
        ym = y * m

        @pl.when(i == 0)
        def _():
            st_ref[...] = jnp.zeros_like(st_ref)

        st_ref[0:1, :] += jnp.sum(ym, axis=0, keepdims=True)
        st_ref[1:2, :] += jnp.sum(ym * ym, axis=0, keepdims=True)

    return pl.pallas_call(
        body,
        grid=(_NPAD // _BR,),
        in_specs=[
            pl.BlockSpec((1, _BR, _D), lambda i: (0, i, 0)),
            pl.BlockSpec((1, _BR, _D), lambda i: (1, i, 0)),
            pl.BlockSpec((_BR, _D), lambda i: (i, 0)),
            pl.BlockSpec((_BR, 1), lambda i: (i, 0)),
            pl.BlockSpec((1, _D), lambda i: (0, 0)),
        ],
        out_specs=[
            pl.BlockSpec((_BR, _D), lambda i: (i, 0)),
            pl.BlockSpec((2, _D), lambda i: (0, 0)),
        ],
        out_shape=[
            jax.ShapeDtypeStruct((_NPAD, _D), jnp.float32),
            jax.ShapeDtypeStruct((2, _D), jnp.float32),
        ],
    )(agg, agg, t, dinv, b)


def _bn_act_mm(y, st, g, bt, a, W, dinv):
    def body(y_ref, st_ref, g_ref, bt_ref, a_ref, w_ref, dv_ref, o_ref):
        st = st_ref[...]
        mu = st[0:1, :] * (1.0 / _N)
        var = st[1:2, :] * (1.0 / _N) - mu * mu
        z = g_ref[...] * (y_ref[...] - mu) / jnp.sqrt(var + _EPS) + bt_ref[...]
        h = jnp.where(z >= 0, z, a_ref[...] * z)
        o_ref[...] = jnp.dot(h, w_ref[...],
                             preferred_element_type=jnp.float32) * dv_ref[...]

    return pl.pallas_call(
        body,
        grid=(_NPAD // _BR,),
        in_specs=[
            pl.BlockSpec((_BR, _D), lambda i: (i, 0)),
            pl.BlockSpec((2, _D), lambda i: (0, 0)),
            pl.BlockSpec((1, _D), lambda i: (0, 0)),
            pl.BlockSpec((1, _D), lambda i: (0, 0)),
            pl.BlockSpec((1, 1), lambda i: (0, 0)),
            pl.BlockSpec((_D, _D), lambda i: (0, 0)),
            pl.BlockSpec((_BR, 1), lambda i: (i, 0)),
        ],
        out_specs=pl.BlockSpec((_BR, _D), lambda i: (i, 0)),
        out_shape=jax.ShapeDtypeStruct((_NPAD, _D), jnp.float32),
    )(y, st, g, bt, a, W, dinv)


def _bn_act(y, st, g, bt, a):
    def body(y_ref, st_ref, g_ref, bt_ref, a_ref, o_ref):
        st = st_ref[...]
        mu = st[0:1, :] * (1.0 / _N)
        var = st[1:2, :] * (1.0 / _N) - mu * mu
        z = g_ref[...] * (y_ref[...] - mu) / jnp.sqrt(var + _EPS) + bt_ref[...]
        o_ref[...] = jnp.where(z >= 0, z, a_ref[...] * z)

    return pl.pallas_call(
        body,
        grid=(_NPAD // _BR,),
        in_specs=[
            pl.BlockSpec((_BR, _D), lambda i: (i, 0)),
            pl.BlockSpec((2, _D), lambda i: (0, 0)),
            pl.BlockSpec((1, _D), lambda i: (0, 0)),
            pl.BlockSpec((1, _D), lambda i: (0, 0)),
            pl.BlockSpec((1, 1), lambda i: (0, 0)),
        ],
        out_specs=pl.BlockSpec((_BR, _D), lambda i: (i, 0)),
        out_shape=jax.ShapeDtypeStruct((_NPAD, _D), jnp.float32),
    )(y, st, g, bt, a)


def kernel(x, edge_index, W1, b1, g1, bt1, a1, W2, b2, g2, bt2, a2):
    src = edge_index[0]
    dst = edge_index[1]
    pad = _EPAD - _E
    src_p = jnp.concatenate(
        [src, jnp.zeros((pad,), jnp.int32)]).reshape(_NW, _CH, _C)
    dst_p = jnp.concatenate(
        [dst, jnp.full((pad,), _N, jnp.int32)]).reshape(_NW, _CH, _C)
    x_p = jnp.pad(x, ((0, _NPAD - _N), (0, 0)))
    onesD = jnp.ones((_C, _D), jnp.float32)
    zerosRT = jnp.zeros((_RT, _D), jnp.float32)

    degp = _deg_kernel(dst_p, onesD, zerosRT)
    dinv = _dinv_kernel(degp)

    t1 = _mm_scale(x_p, W1, dinv)
    agg1 = _agg_kernel(t1, src_p, dst_p, zerosRT)
    y1, st1 = _combine_stats(agg1, t1, dinv, b1.reshape(1, _D))
    t2 = _bn_act_mm(y1, st1, g1.reshape(1, _D), bt1.reshape(1, _D),
                    a1.reshape(1, 1), W2, dinv)
    agg2 = _agg_kernel(t2, src_p, dst_p, zerosRT)
    y2, st2 = _combine_stats(agg2, t2, dinv, b2.reshape(1, _D))
    h = _bn_act(y2, st2, g2.reshape(1, _D), bt2.reshape(1, _D),
                a2.reshape(1, 1))
    return h[:_N]


# speedup vs baseline: 8.7476x; 1.0706x over previous
import functools

import jax
import jax.numpy as jnp
from jax import lax
from jax.experimental import pallas as pl
from jax.experimental.pallas import tpu as pltpu
from jax.experimental.pallas import tpu_sc as plsc

_N = 10000
_D = 128
_E = 320000
_NPAD = 10240
_EPAD = 327680
_NW = 32
_CH = 80
_C = 128
_RT = _NPAD // 16
_EPS = 1e-5
_BR = 512

_mesh = plsc.VectorSubcoreMesh(core_axis_name="c", subcore_axis_name="s")


@functools.partial(
    pl.kernel,
    out_type=jax.ShapeDtypeStruct((2, _NPAD, _D), jnp.float32),
    mesh=_mesh,
    scratch_types=[
        pltpu.VMEM((_CH, _C), jnp.int32),
        pltpu.VMEM((_C, _D), jnp.float32),
        pltpu.VMEM_SHARED((_NPAD, _D), jnp.float32),
    ],
)
def _deg_kernel(dst_hbm, ones_hbm, zeros_hbm, out_hbm, didx, ones_v, deg_sh):
    c = lax.axis_index("c")
    s = lax.axis_index("s")
    wid = s * 2 + c
    pltpu.sync_copy(dst_hbm.at[wid], didx)
    pltpu.sync_copy(ones_hbm, ones_v)
    pltpu.sync_copy(zeros_hbm, deg_sh.at[pl.ds(s * _RT, _RT)])
    plsc.subcore_barrier()

    def body(j, carry):
        pltpu.sync_copy(ones_v, deg_sh.at[didx.at[j]], add=True)
        return carry

    lax.fori_loop(0, _CH, body, 0)
    plsc.subcore_barrier()
    pltpu.sync_copy(deg_sh.at[pl.ds(s * _RT, _RT)],
                    out_hbm.at[c, pl.ds(s * _RT, _RT)])


@functools.partial(
    pl.kernel,
    out_type=jax.ShapeDtypeStruct((2, _NPAD, _D), jnp.float32),
    mesh=_mesh,
    scratch_types=[
        pltpu.VMEM((_CH, _C), jnp.int32),
        pltpu.VMEM((_C,), jnp.int32),
        pltpu.VMEM((_C,), jnp.int32),
        pltpu.VMEM((_C, _D), jnp.float32),
        pltpu.VMEM((_C, _D), jnp.float32),
        pltpu.VMEM_SHARED((_NPAD, _D), jnp.float32),
        pltpu.SemaphoreType.DMA,
        pltpu.SemaphoreType.DMA,
        pltpu.SemaphoreType.DMA,
        pltpu.SemaphoreType.DMA,
        pltpu.SemaphoreType.DMA,
        pltpu.SemaphoreType.DMA,
    ],
)
def _agg_kernel(t_hbm, src_hbm, dst_hbm, zeros_hbm, out_hbm, didx, si0, si1,
                rows0, rows1, agg_sh, semI0, semI1, semG0, semG1, semS0,
                semS1):
    c = lax.axis_index("c")
    s = lax.axis_index("s")
    wid = s * 2 + c
    pltpu.sync_copy(dst_hbm.at[wid], didx)
    pltpu.sync_copy(zeros_hbm, agg_sh.at[pl.ds(s * _RT, _RT)])
    plsc.subcore_barrier()

    def ld_idx(k, si, semI):
        pltpu.async_copy(src_hbm.at[wid, k], si, semI)

    def wt_idx(k, si, semI):
        pltpu.make_async_copy(src_hbm.at[wid, k], si, semI).wait()

    def gather(si, rows, semG):
        pltpu.async_copy(t_hbm.at[si], rows, semG)

    def wt_gather(si, rows, semG):
        pltpu.make_async_copy(t_hbm.at[si], rows, semG).wait()

    def scatter(k, rows, semS):
        pltpu.async_copy(rows, agg_sh.at[didx.at[k]], semS, add=True)

    def wt_scatter(k, rows, semS):
        pltpu.make_async_copy(rows, agg_sh.at[didx.at[k]], semS).wait()

    ld_idx(0, si0, semI0)
    ld_idx(1, si1, semI1)
    wt_idx(0, si0, semI0)
    gather(si0, rows0, semG0)
    wt_gather(si0, rows0, semG0)
    ld_idx(2, si0, semI0)
    scatter(0, rows0, semS0)
    wt_idx(1, si1, semI1)
    gather(si1, rows1, semG1)

    def body(jj, carry):
        ka = 2 * jj + 1
        wt_gather(si1, rows1, semG1)
        ld_idx(ka + 2, si1, semI1)
        scatter(ka, rows1, semS1)
        wt_idx(ka + 1, si0, semI0)
        wt_scatter(ka - 1, rows0, semS0)
        gather(si0, rows0, semG0)
        kb = ka + 1
        wt_gather(si0, rows0, semG0)
        ld_idx(kb + 2, si0, semI0)
        scatter(kb, rows0, semS0)
        wt_idx(kb + 1, si1, semI1)
        wt_scatter(kb - 1, rows1, semS1)
        gather(si1, rows1, semG1)
        return carry

    lax.fori_loop(0, (_CH - 4) // 2, body, 0)

    wt_gather(si1, rows1, semG1)
    ld_idx(79, si1, semI1)
    scatter(77, rows1, semS1)
    wt_idx(78, si0, semI0)
    wt_scatter(76, rows0, semS0)
    gather(si0, rows0, semG0)

    wt_gather(si0, rows0, semG0)
    scatter(78, rows0, semS0)
    wt_idx(79, si1, semI1)
    wt_scatter(77, rows1, semS1)
    gather(si1, rows1, semG1)

    wt_gather(si1, rows1, semG1)
    scatter(79, rows1, semS1)
    wt_scatter(78, rows0, semS0)
    wt_scatter(79, rows1, semS1)

    plsc.subcore_barrier()
    pltpu.sync_copy(agg_sh.at[pl.ds(s * _RT, _RT)],
                    out_hbm.at[c, pl.ds(s * _RT, _RT)])


def _dinv_kernel(degp):
    def body(d_ref, o_ref):
        d = d_ref[...]
        deg = d[0, :, 0:1] + d[1, :, 0:1] + 1.0
        o_ref[...] = 1.0 / jnp.sqrt(deg)

    return pl.pallas_call(
        body,
        out_shape=jax.ShapeDtypeStruct((_NPAD, 1), jnp.float32),
    )(degp)


def _mm_scale(x, W, dinv):
    def body(x_ref, w_ref, dv_ref, o_ref):
        o_ref[...] = jnp.dot(x_ref[...], w_ref[...],
                             preferred_element_type=jnp.float32) * dv_ref[...]

    return pl.pallas_call(
        body,
        grid=(_NPAD // _BR,),
        in_specs=[
            pl.BlockSpec((_BR, _D), lambda i: (i, 0)),
            pl.BlockSpec((_D, _D), lambda i: (0, 0)),
            pl.BlockSpec((_BR, 1), lambda i: (i, 0)),
        ],
        out_specs=pl.BlockSpec((_BR, _D), lambda i: (i, 0)),
        out_shape=jax.ShapeDtypeStruct((_NPAD, _D), jnp.float32),
    )(x, W, dinv)


def _combine_stats(agg, t, dinv, b):
    def body(a0_ref, a1_ref, t_ref, dv_ref, b_ref, y_ref, st_ref):
        i = pl.program_id(0)
        y = dv_ref[...] * (a0_ref[...][0] + a1_ref[...][0] + t_ref[...]) \
            + b_ref[...]
        y_ref[...] = y
        rid = lax.broadcasted_iota(jnp.int32, (_BR, 1), 0) + i * _BR
        m = (rid < _N).astype(jnp.float32)
        ym = y * m

        @pl.when(i == 0)
        def _():
            st_ref[...] = jnp.zeros_like(st_ref)

        st_ref[0:1, :] += jnp.sum(ym, axis=0, keepdims=True)
        st_ref[1:2, :] += jnp.sum(ym * ym, axis=0, keepdims=True)

    return pl.pallas_call(
        body,
        grid=(_NPAD // _BR,),
        in_specs=[
            pl.BlockSpec((1, _BR, _D), lambda i: (0, i, 0)),
            pl.BlockSpec((1, _BR, _D), lambda i: (1, i, 0)),
            pl.BlockSpec((_BR, _D), lambda i: (i, 0)),
            pl.BlockSpec((_BR, 1), lambda i: (i, 0)),
            pl.BlockSpec((1, _D), lambda i: (0, 0)),
        ],
        out_specs=[
            pl.BlockSpec((_BR, _D), lambda i: (i, 0)),
            pl.BlockSpec((2, _D), lambda i: (0, 0)),
        ],
        out_shape=[
            jax.ShapeDtypeStruct((_NPAD, _D), jnp.float32),
            jax.ShapeDtypeStruct((2, _D), jnp.float32),
        ],
    )(agg, agg, t, dinv, b)


def _bn_act_mm(y, st, g, bt, a, W, dinv):
    def body(y_ref, st_ref, g_ref, bt_ref, a_ref, w_ref, dv_ref, o_ref):
        st = st_ref[...]
        mu = st[0:1, :] * (1.0 / _N)
        var = st[1:2, :] * (1.0 / _N) - mu * mu
        z = g_ref[...] * (y_ref[...] - mu) / jnp.sqrt(var + _EPS) + bt_ref[...]
        h = jnp.where(z >= 0, z, a_ref[...] * z)
        o_ref[...] = jnp.dot(h, w_ref[...],
                             preferred_element_type=jnp.float32) * dv_ref[...]

    return pl.pallas_call(
        body,
        grid=(_NPAD // _BR,),
        in_specs=[
            pl.BlockSpec((_BR, _D), lambda i: (i, 0)),
            pl.BlockSpec((2, _D), lambda i: (0, 0)),
            pl.BlockSpec((1, _D), lambda i: (0, 0)),
            pl.BlockSpec((1, _D), lambda i: (0, 0)),
            pl.BlockSpec((1, 1), lambda i: (0, 0)),
            pl.BlockSpec((_D, _D), lambda i: (0, 0)),
            pl.BlockSpec((_BR, 1), lambda i: (i, 0)),
        ],
        out_specs=pl.BlockSpec((_BR, _D), lambda i: (i, 0)),
        out_shape=jax.ShapeDtypeStruct((_NPAD, _D), jnp.float32),
    )(y, st, g, bt, a, W, dinv)


def _bn_act(y, st, g, bt, a):
    def body(y_ref, st_ref, g_ref, bt_ref, a_ref, o_ref):
        st = st_ref[...]
        mu = st[0:1, :] * (1.0 / _N)
        var = st[1:2, :] * (1.0 / _N) - mu * mu
        z = g_ref[...] * (y_ref[...] - mu) / jnp.sqrt(var + _EPS) + bt_ref[...]
        o_ref[...] = jnp.where(z >= 0, z, a_ref[...] * z)

    return pl.pallas_call(
        body,
        grid=(_NPAD // _BR,),
        in_specs=[
            pl.BlockSpec((_BR, _D), lambda i: (i, 0)),
            pl.BlockSpec((2, _D), lambda i: (0, 0)),
            pl.BlockSpec((1, _D), lambda i: (0, 0)),
            pl.BlockSpec((1, _D), lambda i: (0, 0)),
            pl.BlockSpec((1, 1), lambda i: (0, 0)),
        ],
        out_specs=pl.BlockSpec((_BR, _D), lambda i: (i, 0)),
        out_shape=jax.ShapeDtypeStruct((_NPAD, _D), jnp.float32),
    )(y, st, g, bt, a)


def kernel(x, edge_index, W1, b1, g1, bt1, a1, W2, b2, g2, bt2, a2):
    src = edge_index[0]
    dst = edge_index[1]
    pad = _EPAD - _E
    src_p = jnp.concatenate(
        [src, jnp.zeros((pad,), jnp.int32)]).reshape(_NW, _CH, _C)
    dst_p = jnp.concatenate(
        [dst, jnp.full((pad,), _N, jnp.int32)]).reshape(_NW, _CH, _C)
    x_p = jnp.pad(x, ((0, _NPAD - _N), (0, 0)))
    onesD = jnp.ones((_C, _D), jnp.float32)
    zerosRT = jnp.zeros((_RT, _D), jnp.float32)

    degp = _deg_kernel(dst_p, onesD, zerosRT)
    dinv = _dinv_kernel(degp)

    t1 = _mm_scale(x_p, W1, dinv)
    agg1 = _agg_kernel(t1, src_p, dst_p, zerosRT)
    y1, st1 = _combine_stats(agg1, t1, dinv, b1.reshape(1, _D))
    t2 = _bn_act_mm(y1, st1, g1.reshape(1, _D), bt1.reshape(1, _D),
                    a1.reshape(1, 1), W2, dinv)
    agg2 = _agg_kernel(t2, src_p, dst_p, zerosRT)
    y2, st2 = _combine_stats(agg2, t2, dinv, b2.reshape(1, _D))
    h = _bn_act(y2, st2, g2.reshape(1, _D), bt2.reshape(1, _D),
                a2.reshape(1, 1))
    return h[:_N]


# speedup vs baseline: 21.4121x; 2.4478x over previous
import functools

import jax
import jax.numpy as jnp
from jax import lax
from jax.experimental import pallas as pl
from jax.experimental.pallas import tpu as pltpu
from jax.experimental.pallas import tpu_sc as plsc

_N = 10000
_D = 128
_E = 320000
_NPAD = 10240
_EPAD = 327680
_NW = 32
_CH = 80
_C = 128
_RT = _NPAD // 16
_EPS = 1e-5
_BR = 512

_mesh = plsc.VectorSubcoreMesh(core_axis_name="c", subcore_axis_name="s")


@functools.partial(
    pl.kernel,
    out_type=jax.ShapeDtypeStruct((2, _NPAD, _D), jnp.float32),
    mesh=_mesh,
    scratch_types=[
        pltpu.VMEM((_CH, _C), jnp.int32),
        pltpu.VMEM((_C, _D), jnp.float32),
        pltpu.VMEM_SHARED((_NPAD, _D), jnp.float32),
    ],
)
def _deg_kernel(dst_hbm, ones_hbm, zeros_hbm, out_hbm, didx, ones_v, deg_sh):
    c = lax.axis_index("c")
    s = lax.axis_index("s")
    wid = s * 2 + c
    pltpu.sync_copy(dst_hbm.at[wid], didx)
    pltpu.sync_copy(ones_hbm, ones_v)
    pltpu.sync_copy(zeros_hbm, deg_sh.at[pl.ds(s * _RT, _RT)])
    plsc.subcore_barrier()

    def body(j, carry):
        pltpu.sync_copy(ones_v, deg_sh.at[didx.at[j]], add=True)
        return carry

    lax.fori_loop(0, _CH, body, 0)
    plsc.subcore_barrier()
    pltpu.sync_copy(deg_sh.at[pl.ds(s * _RT, _RT)],
                    out_hbm.at[c, pl.ds(s * _RT, _RT)])


@functools.partial(
    pl.kernel,
    out_type=jax.ShapeDtypeStruct((2, _NPAD, _D), jnp.float32),
    mesh=_mesh,
    scratch_types=[
        pltpu.VMEM((_CH, _C), jnp.int32),
        pltpu.VMEM((_C,), jnp.int32),
        pltpu.VMEM((_C,), jnp.int32),
        pltpu.VMEM((_C, _D), jnp.float32),
        pltpu.VMEM((_C, _D), jnp.float32),
        pltpu.VMEM_SHARED((_NPAD, _D), jnp.float32),
        pltpu.SemaphoreType.DMA,
        pltpu.SemaphoreType.DMA,
        pltpu.SemaphoreType.DMA,
        pltpu.SemaphoreType.DMA,
        pltpu.SemaphoreType.DMA,
        pltpu.SemaphoreType.DMA,
    ],
)
def _agg_kernel(t_hbm, src_hbm, dst_hbm, zeros_hbm, out_hbm, didx, si0, si1,
                rows0, rows1, agg_sh, semI0, semI1, semG0, semG1, semS0,
                semS1):
    c = lax.axis_index("c")
    s = lax.axis_index("s")
    wid = s * 2 + c
    pltpu.sync_copy(dst_hbm.at[wid], didx)
    pltpu.sync_copy(zeros_hbm, agg_sh.at[pl.ds(s * _RT, _RT)])
    plsc.subcore_barrier()

    def ld_idx(k, si, semI):
        pltpu.async_copy(src_hbm.at[wid, k], si, semI)

    def wt_idx(k, si, semI):
        pltpu.make_async_copy(src_hbm.at[wid, k], si, semI).wait()

    def gather(si, rows, semG):
        pltpu.async_copy(t_hbm.at[si], rows, semG)

    def wt_gather(si, rows, semG):
        pltpu.make_async_copy(t_hbm.at[si], rows, semG).wait()

    def scatter(k, rows, semS):
        pltpu.async_copy(rows, agg_sh.at[didx.at[k]], semS, add=True)

    def wt_scatter(k, rows, semS):
        pltpu.make_async_copy(rows, agg_sh.at[didx.at[k]], semS).wait()

    ld_idx(0, si0, semI0)
    ld_idx(1, si1, semI1)
    wt_idx(0, si0, semI0)
    gather(si0, rows0, semG0)
    wt_gather(si0, rows0, semG0)
    ld_idx(2, si0, semI0)
    scatter(0, rows0, semS0)
    wt_idx(1, si1, semI1)
    gather(si1, rows1, semG1)

    def body(jj, carry):
        ka = 2 * jj + 1
        wt_gather(si1, rows1, semG1)
        ld_idx(ka + 2, si1, semI1)
        scatter(ka, rows1, semS1)
        wt_idx(ka + 1, si0, semI0)
        wt_scatter(ka - 1, rows0, semS0)
        gather(si0, rows0, semG0)
        kb = ka + 1
        wt_gather(si0, rows0, semG0)
        ld_idx(kb + 2, si0, semI0)
        scatter(kb, rows0, semS0)
        wt_idx(kb + 1, si1, semI1)
        wt_scatter(kb - 1, rows1, semS1)
        gather(si1, rows1, semG1)
        return carry

    lax.fori_loop(0, (_CH - 4) // 2, body, 0)

    wt_gather(si1, rows1, semG1)
    ld_idx(79, si1, semI1)
    scatter(77, rows1, semS1)
    wt_idx(78, si0, semI0)
    wt_scatter(76, rows0, semS0)
    gather(si0, rows0, semG0)

    wt_gather(si0, rows0, semG0)
    scatter(78, rows0, semS0)
    wt_idx(79, si1, semI1)
    wt_scatter(77, rows1, semS1)
    gather(si1, rows1, semG1)

    wt_gather(si1, rows1, semG1)
    scatter(79, rows1, semS1)
    wt_scatter(78, rows0, semS0)
    wt_scatter(79, rows1, semS1)

    plsc.subcore_barrier()
    pltpu.sync_copy(agg_sh.at[pl.ds(s * _RT, _RT)],
                    out_hbm.at[c, pl.ds(s * _RT, _RT)])


def _dinv_kernel(degp):
    def body(d_ref, o_ref):
        d = d_ref[...]
        deg = d[0, :, 0:1] + d[1, :, 0:1] + 1.0
        o_ref[...] = 1.0 / jnp.sqrt(deg)

    return pl.pallas_call(
        body,
        out_shape=jax.ShapeDtypeStruct((_NPAD, 1), jnp.float32),
    )(degp)


def _mm_scale(x, W, dinv):
    def body(x_ref, w_ref, dv_ref, o_ref):
        o_ref[...] = jnp.dot(x_ref[...], w_ref[...],
                             preferred_element_type=jnp.float32) * dv_ref[...]

    return pl.pallas_call(
        body,
        grid=(_NPAD // _BR,),
        in_specs=[
            pl.BlockSpec((_BR, _D), lambda i: (i, 0)),
            pl.BlockSpec((_D, _D), lambda i: (0, 0)),
            pl.BlockSpec((_BR, 1), lambda i: (i, 0)),
        ],
        out_specs=pl.BlockSpec((_BR, _D), lambda i: (i, 0)),
        out_shape=jax.ShapeDtypeStruct((_NPAD, _D), jnp.float32),
    )(x, W, dinv)


def _combine_stats(agg, t, dinv, b):
    def body(a0_ref, a1_ref, t_ref, dv_ref, b_ref, y_ref, st_ref):
        i = pl.program_id(0)
        y = dv_ref[...] * (a0_ref[...][0] + a1_ref[...][0] + t_ref[...]) \
            + b_ref[...]
        y_ref[...] = y
        rid = lax.broadcasted_iota(jnp.int32, (_BR, 1), 0) + i * _BR
        m = (rid < _N).astype(jnp.float32)
        ym = y * m

        @pl.when(i == 0)
        def _():
            st_ref[...] = jnp.zeros_like(st_ref)

        st_ref[0:1, :] += jnp.sum(ym, axis=0, keepdims=True)
        st_ref[1:2, :] += jnp.sum(ym * ym, axis=0, keepdims=True)

    return pl.pallas_call(
        body,
        grid=(_NPAD // _BR,),
        in_specs=[
            pl.BlockSpec((1, _BR, _D), lambda i: (0, i, 0)),
            pl.BlockSpec((1, _BR, _D), lambda i: (1, i, 0)),
            pl.BlockSpec((_BR, _D), lambda i: (i, 0)),
            pl.BlockSpec((_BR, 1), lambda i: (i, 0)),
            pl.BlockSpec((1, _D), lambda i: (0, 0)),
        ],
        out_specs=[
            pl.BlockSpec((_BR, _D), lambda i: (i, 0)),
            pl.BlockSpec((2, _D), lambda i: (0, 0)),
        ],
        out_shape=[
            jax.ShapeDtypeStruct((_NPAD, _D), jnp.float32),
            jax.ShapeDtypeStruct((2, _D), jnp.float32),
        ],
    )(agg, agg, t, dinv, b)


def _bn_act_mm(y, st, g, bt, a, W, dinv):
    def body(y_ref, st_ref, g_ref, bt_ref, a_ref, w_ref, dv_ref, o_ref):
        st = st_ref[...]
        mu = st[0:1, :] * (1.0 / _N)
        var = st[1:2, :] * (1.0 / _N) - mu * mu
        z = g_ref[...] * (y_ref[...] - mu) / jnp.sqrt(var + _EPS) + bt_ref[...]
        h = jnp.where(z >= 0, z, a_ref[...] * z)
        o_ref[...] = jnp.dot(h, w_ref[...],
                             preferred_element_type=jnp.float32) * dv_ref[...]

    return pl.pallas_call(
        body,
        grid=(_NPAD // _BR,),
        in_specs=[
            pl.BlockSpec((_BR, _D), lambda i: (i, 0)),
            pl.BlockSpec((2, _D), lambda i: (0, 0)),
            pl.BlockSpec((1, _D), lambda i: (0, 0)),
            pl.BlockSpec((1, _D), lambda i: (0, 0)),
            pl.BlockSpec((1, 1), lambda i: (0, 0)),
            pl.BlockSpec((_D, _D), lambda i: (0, 0)),
            pl.BlockSpec((_BR, 1), lambda i: (i, 0)),
        ],
        out_specs=pl.BlockSpec((_BR, _D), lambda i: (i, 0)),
        out_shape=jax.ShapeDtypeStruct((_NPAD, _D), jnp.float32),
    )(y, st, g, bt, a, W, dinv)


def _bn_act(y, st, g, bt, a):
    def body(y_ref, st_ref, g_ref, bt_ref, a_ref, o_ref):
        st = st_ref[...]
        mu = st[0:1, :] * (1.0 / _N)
        var = st[1:2, :] * (1.0 / _N) - mu * mu
        z = g_ref[...] * (y_ref[...] - mu) / jnp.sqrt(var + _EPS) + bt_ref[...]
        o_ref[...] = jnp.where(z >= 0, z, a_ref[...] * z)

    return pl.pallas_call(
        body,
        grid=(_NPAD // _BR,),
        in_specs=[
            pl.BlockSpec((_BR, _D), lambda i: (i, 0)),
            pl.BlockSpec((2, _D), lambda i: (0, 0)),
            pl.BlockSpec((1, _D), lambda i: (0, 0)),
            pl.BlockSpec((1, _D), lambda i: (0, 0)),
            pl.BlockSpec((1, 1), lambda i: (0, 0)),
        ],
        out_specs=pl.BlockSpec((_BR, _D), lambda i: (i, 0)),
        out_shape=jax.ShapeDtypeStruct((_NPAD, _D), jnp.float32),
    )(y, st, g, bt, a)


def kernel(x, edge_index, W1, b1, g1, bt1, a1, W2, b2, g2, bt2, a2):
    src = edge_index[0]
    dst = edge_index[1]
    pad = _EPAD - _E
    pad_src = (jnp.arange(pad, dtype=jnp.int32) * 13) % _N
    src_p = jnp.concatenate([src, pad_src]).reshape(_NW, _CH, _C)
    dst_p = jnp.concatenate(
        [dst, jnp.full((pad,), _N, jnp.int32)]).reshape(_NW, _CH, _C)
    x_p = jnp.pad(x, ((0, _NPAD - _N), (0, 0)))
    onesD = jnp.ones((_C, _D), jnp.float32)
    zerosRT = jnp.zeros((_RT, _D), jnp.float32)

    degp = _deg_kernel(dst_p, onesD, zerosRT)
    dinv = _dinv_kernel(degp)

    t1 = _mm_scale(x_p, W1, dinv)
    agg1 = _agg_kernel(t1, src_p, dst_p, zerosRT)
    y1, st1 = _combine_stats(agg1, t1, dinv, b1.reshape(1, _D))
    t2 = _bn_act_mm(y1, st1, g1.reshape(1, _D), bt1.reshape(1, _D),
                    a1.reshape(1, 1), W2, dinv)
    agg2 = _agg_kernel(t2, src_p, dst_p, zerosRT)
    y2, st2 = _combine_stats(agg2, t2, dinv, b2.reshape(1, _D))
    h = _bn_act(y2, st2, g2.reshape(1, _D), bt2.reshape(1, _D),
                a2.reshape(1, 1))
    return h[:_N]
